# baseline (speedup 1.0000x reference)
import functools

import jax
import jax.numpy as jnp
from jax import lax
from jax.experimental import pallas as pl
from jax.experimental.pallas import tpu as pltpu
from jax.experimental.pallas import tpu_sc as plsc

B = 16384
VOCAB = 100000
EMB = 50
EMBP = 64
NPOS = 48
G = NPOS * EMBP // 128
H_UNITS = 256
NUM_CLASSES = 128

NC, NS = 2, 16
NW = NC * NS
CH = 128
NSLICE = 4
BS = B // NSLICE


@functools.lru_cache(maxsize=None)
def _make_sc_gather():
    mesh = plsc.VectorSubcoreMesh(
        core_axis_name="c", subcore_axis_name="s", num_cores=NC, num_subcores=NS
    )
    rpw = BS // NW
    nch = rpw // CH

    @functools.partial(
        pl.kernel,
        out_type=jax.ShapeDtypeStruct((G, BS, 128), jnp.bfloat16),
        mesh=mesh,
        scratch_types=[
            pltpu.VMEM((NPOS, CH), jnp.int32),
            pltpu.VMEM((2, CH, EMBP), jnp.bfloat16),
            pltpu.SemaphoreType.DMA,
            pltpu.SemaphoreType.DMA,
        ],
        compiler_params=pltpu.CompilerParams(use_tc_tiling_on_sc=False),
    )
    def _sc_gather(xT_hbm, w_word, w_pos, w_lab, out_hbm, idx_v, buf, gsem, wsem):
        wid = lax.axis_index("s") * NC + lax.axis_index("c")
        base = wid * rpw

        def _gather(table, j, slot):
            pltpu.async_copy(table.at[idx_v.at[j]], buf.at[slot], gsem)

        def _wait_gather(table, slot):
            pltpu.make_async_copy(table.at[idx_v.at[0]], buf.at[slot], gsem).wait()

        def _dst(rbase, j):
            return out_hbm.at[
                lax.div(j, 2), pl.ds(rbase, CH), pl.ds(lax.rem(j, 2) * EMBP, EMBP)
            ]

        def _write(rbase, j, slot):
            pltpu.async_copy(buf.at[slot], _dst(rbase, j), wsem)

        def _wait_write(rbase, slot):
            pltpu.make_async_copy(buf.at[slot], _dst(rbase, 0), wsem).wait()

        @pl.loop(0, nch)
        def _chunk(c):
            rbase = base + c * CH
            pltpu.sync_copy(xT_hbm.at[:, pl.ds(rbase, CH)], idx_v)
            for table, j0, nj in ((w_word, 0, 18), (w_pos, 18, 18), (w_lab, 36, 12)):
                _gather(table, j0, 0)

                @pl.loop(0, nj)
                def _pos(jj, table=table, j0=j0, rbase=rbase, nj=nj):
                    j = j0 + jj
                    cur = lax.rem(jj, 2)

                    _wait_gather(table, cur)
                    _write(rbase, j, cur)

                    @pl.when(jj > 0)
                    def _():
                        _wait_write(rbase, 1 - cur)

                    @pl.when(jj < nj - 1)
                    def _():
                        _gather(table, j + 1, 1 - cur)

                _wait_write(rbase, lax.rem(nj - 1, 2))

    return _sc_gather


BM = 512


def _mlp_body(x_ref, w1_ref, b1_ref, w2_ref, b2_ref, o_ref):
    x = x_ref[...]
    prods = jax.lax.dot_general(
        x,
        w1_ref[...],
        (((2,), (1,)), ((0,), (0,))),
        preferred_element_type=jnp.float32,
    )
    acc = jnp.sum(prods, axis=0) + b1_ref[...]
    h = jnp.tanh(acc)
    o_ref[...] = (
        jnp.dot(h, w2_ref[...], preferred_element_type=jnp.float32) + b2_ref[...]
    )


_mlp = pl.pallas_call(
    _mlp_body,
    grid=(BS // BM,),
    in_specs=[
        pl.BlockSpec((G, BM, 128), lambda i: (0, i, 0)),
        pl.BlockSpec((G, 128, H_UNITS), lambda i: (0, 0, 0)),
        pl.BlockSpec((1, H_UNITS), lambda i: (0, 0)),
        pl.BlockSpec((H_UNITS, NUM_CLASSES), lambda i: (0, 0)),
        pl.BlockSpec((1, NUM_CLASSES), lambda i: (0, 0)),
    ],
    out_specs=pl.BlockSpec((BM, NUM_CLASSES), lambda i: (i, 0)),
    out_shape=jax.ShapeDtypeStruct((BS, NUM_CLASSES), jnp.float32),
)


def kernel(input_data, W_word, W_pos, W_label, W1, b1, W2, b2):
    x = input_data
    if x.shape[0] == 1:
        x = x[0]
    xT = x.T
    pad = ((0, 0), (0, EMBP - EMB))
    tabs = [
        jnp.pad(t.astype(jnp.bfloat16), pad) for t in (W_word, W_pos, W_label)
    ]
    w1b = (
        jnp.pad(W1.reshape(NPOS, EMB, H_UNITS), ((0, 0), (0, EMBP - EMB), (0, 0)))
        .reshape(G, 128, H_UNITS)
        .astype(jnp.bfloat16)
    )
    b1r, b2r = b1.reshape(1, -1), b2.reshape(1, -1)
    gather = _make_sc_gather()
    outs = []
    for s in range(NSLICE):
        alldata = gather(xT[:, s * BS : (s + 1) * BS], *tabs)
        outs.append(_mlp(alldata, w1b, b1r, W2, b2r))
    return jnp.concatenate(outs, axis=0)


# speedup vs baseline: 2.0744x; 2.0744x over previous
import functools

import jax
import jax.numpy as jnp
from jax import lax
from jax.experimental import pallas as pl
from jax.experimental.pallas import tpu as pltpu
from jax.experimental.pallas import tpu_sc as plsc

B = 16384
VOCAB = 100000
EMB = 50
EMBP = 64
NPOS = 48
G = NPOS * EMBP // 128
H_UNITS = 256
NUM_CLASSES = 128

NC, NS = 2, 16
NW = NC * NS
CH = 128
NSLICE = 8
BS = B // NSLICE


@functools.lru_cache(maxsize=None)
def _make_sc_gather():
    mesh = plsc.VectorSubcoreMesh(
        core_axis_name="c", subcore_axis_name="s", num_cores=NC, num_subcores=NS
    )
    rpw = BS // NW
    nch = rpw // CH

    @functools.partial(
        pl.kernel,
        out_type=jax.ShapeDtypeStruct((G, BS, 128), jnp.float32),
        mesh=mesh,
        scratch_types=[
            pltpu.VMEM((NPOS, CH), jnp.int32),
            pltpu.VMEM((2, CH, EMBP), jnp.float32),
            pltpu.SemaphoreType.DMA,
            pltpu.SemaphoreType.DMA,
        ],
        compiler_params=pltpu.CompilerParams(use_tc_tiling_on_sc=False),
    )
    def _sc_gather(xT_hbm, w_word, w_pos, w_lab, out_hbm, idx_v, buf, gsem, wsem):
        wid = lax.axis_index("s") * NC + lax.axis_index("c")
        base = wid * rpw

        def _gather(table, j, slot):
            pltpu.async_copy(table.at[idx_v.at[j]], buf.at[slot], gsem)

        def _wait_gather(table, slot):
            pltpu.make_async_copy(table.at[idx_v.at[0]], buf.at[slot], gsem).wait()

        def _dst(rbase, j):
            return out_hbm.at[
                lax.div(j, 2), pl.ds(rbase, CH), pl.ds(lax.rem(j, 2) * EMBP, EMBP)
            ]

        def _write(rbase, j, slot):
            pltpu.async_copy(buf.at[slot], _dst(rbase, j), wsem)

        def _wait_write(rbase, slot):
            pltpu.make_async_copy(buf.at[slot], _dst(rbase, 0), wsem).wait()

        @pl.loop(0, nch)
        def _chunk(c):
            rbase = base + c * CH
            pltpu.sync_copy(xT_hbm.at[:, pl.ds(rbase, CH)], idx_v)
            for table, j0, nj in ((w_word, 0, 18), (w_pos, 18, 18), (w_lab, 36, 12)):
                _gather(table, j0, 0)

                @pl.loop(0, nj)
                def _pos(jj, table=table, j0=j0, rbase=rbase, nj=nj):
                    j = j0 + jj
                    cur = lax.rem(jj, 2)

                    _wait_gather(table, cur)
                    _write(rbase, j, cur)

                    @pl.when(jj > 0)
                    def _():
                        _wait_write(rbase, 1 - cur)

                    @pl.when(jj < nj - 1)
                    def _():
                        _gather(table, j + 1, 1 - cur)

                _wait_write(rbase, lax.rem(nj - 1, 2))

    return _sc_gather


BM = 512


def _mlp_body(x_ref, w1_ref, b1_ref, w2_ref, b2_ref, o_ref):
    x = x_ref[...].astype(jnp.bfloat16)
    prods = jax.lax.dot_general(
        x,
        w1_ref[...],
        (((2,), (1,)), ((0,), (0,))),
        preferred_element_type=jnp.float32,
    )
    acc = jnp.sum(prods, axis=0) + b1_ref[...]
    h = jnp.tanh(acc)
    o_ref[...] = (
        jnp.dot(h, w2_ref[...], preferred_element_type=jnp.float32) + b2_ref[...]
    )


_mlp = pl.pallas_call(
    _mlp_body,
    grid=(BS // BM,),
    in_specs=[
        pl.BlockSpec((G, BM, 128), lambda i: (0, i, 0)),
        pl.BlockSpec((G, 128, H_UNITS), lambda i: (0, 0, 0)),
        pl.BlockSpec((1, H_UNITS), lambda i: (0, 0)),
        pl.BlockSpec((H_UNITS, NUM_CLASSES), lambda i: (0, 0)),
        pl.BlockSpec((1, NUM_CLASSES), lambda i: (0, 0)),
    ],
    out_specs=pl.BlockSpec((BM, NUM_CLASSES), lambda i: (i, 0)),
    out_shape=jax.ShapeDtypeStruct((BS, NUM_CLASSES), jnp.float32),
)


def kernel(input_data, W_word, W_pos, W_label, W1, b1, W2, b2):
    x = input_data
    if x.shape[0] == 1:
        x = x[0]
    xT = x.T
    pad = ((0, 0), (0, EMBP - EMB))
    tabs = [jnp.pad(t, pad) for t in (W_word, W_pos, W_label)]
    w1b = (
        jnp.pad(W1.reshape(NPOS, EMB, H_UNITS), ((0, 0), (0, EMBP - EMB), (0, 0)))
        .reshape(G, 128, H_UNITS)
        .astype(jnp.bfloat16)
    )
    b1r, b2r = b1.reshape(1, -1), b2.reshape(1, -1)
    gather = _make_sc_gather()
    outs = []
    for s in range(NSLICE):
        alldata = gather(xT[:, s * BS : (s + 1) * BS], *tabs)
        outs.append(_mlp(alldata, w1b, b1r, W2, b2r))
    return jnp.concatenate(outs, axis=0)
